# baseline (device time: 35738 ns/iter reference)
import jax
import jax.numpy as jnp
from jax import lax
from jax.experimental import pallas as pl
from jax.experimental.pallas import tpu as pltpu

N_DEV = 4
B = 2
S_LOC = 128
S_GLOB = N_DEV * S_LOC
D = 512
HQ = 4
DH = 64
HD = HQ * DH
SCALE = 0.125


def kernel(x, Wq, Wk, Wv, Wo):
    my = lax.axis_index("i")
    pos = (my * S_LOC + jnp.arange(S_LOC)).astype(jnp.float32)[:, None]
    inv = (1.0 / (10000.0 ** (jnp.arange(0, DH, 2).astype(jnp.float32) / DH)))
    ang = pos * inv[None, :]
    cos = jnp.repeat(jnp.cos(ang), 2, axis=-1)
    sin = jnp.repeat(jnp.sin(ang), 2, axis=-1)
    cos_t = jnp.tile(cos, (1, HQ))
    sin_t = jnp.tile(sin, (1, HQ))

    def body(x_ref, wq_ref, wk_ref, wv_ref, wo_ref, cos_ref, sin_ref,
             out_ref, q_buf, kv_all, comm_ref, send_sems, recv_sems):
        my_pos = lax.axis_index("i")
        left = (my_pos - 1) % N_DEV
        right = (my_pos + 1) % N_DEV

        barrier_sem = pltpu.get_barrier_semaphore()
        for nbr in (left, right):
            pl.semaphore_signal(barrier_sem, inc=1, device_id=(nbr,),
                                device_id_type=pl.DeviceIdType.MESH)
        pl.semaphore_wait(barrier_sem, 2)

        cos_v = cos_ref[:, :]
        sin_v = sin_ref[:, :]
        lane = lax.broadcasted_iota(jnp.int32, (S_LOC, HD), 1)
        even = (lane % 2) == 0

        def rope(t):
            t_r = jnp.where(even, -jnp.roll(t, -1, axis=1),
                            jnp.roll(t, 1, axis=1))
            return t * cos_v + t_r * sin_v

        for b in range(B):
            xb = x_ref[b, :, :]
            q = jnp.dot(xb, wq_ref[:, :], preferred_element_type=jnp.float32)
            k = jnp.dot(xb, wk_ref[:, :], preferred_element_type=jnp.float32)
            v = jnp.dot(xb, wv_ref[:, :], preferred_element_type=jnp.float32)
            q_buf[b, :, :] = rope(q)
            k_rot = rope(k)
            comm_ref[0, b, :, 0:HD] = k_rot
            comm_ref[0, b, :, HD:2 * HD] = v
            kv_all[b, pl.ds(my_pos * S_LOC, S_LOC), 0:HD] = k_rot
            kv_all[b, pl.ds(my_pos * S_LOC, S_LOC), HD:2 * HD] = v

        for h in range(N_DEV - 1):
            send_slot = h % 2
            recv_slot = (h + 1) % 2
            rdma = pltpu.make_async_remote_copy(
                src_ref=comm_ref.at[send_slot],
                dst_ref=comm_ref.at[recv_slot],
                send_sem=send_sems.at[send_slot],
                recv_sem=recv_sems.at[recv_slot],
                device_id=(right,),
                device_id_type=pl.DeviceIdType.MESH,
            )
            rdma.start()
            rdma.wait()
            origin = (my_pos - h - 1) % N_DEV
            for b in range(B):
                kv_all[b, pl.ds(origin * S_LOC, S_LOC), :] = \
                    comm_ref[recv_slot, b, :, :]

        for b in range(B):
            ctx_parts = []
            for hh in range(HQ):
                q = q_buf[b, :, hh * DH:(hh + 1) * DH]
                k_all = kv_all[b, :, hh * DH:(hh + 1) * DH]
                v_all = kv_all[b, :, HD + hh * DH:HD + (hh + 1) * DH]
                s = lax.dot_general(
                    q, k_all, (((1,), (1,)), ((), ())),
                    preferred_element_type=jnp.float32) * SCALE
                m = jnp.max(s, axis=1, keepdims=True)
                w = jnp.exp(s - m)
                w = w / jnp.sum(w, axis=1, keepdims=True)
                ctx_parts.append(
                    jnp.dot(w, v_all, preferred_element_type=jnp.float32))
            ctx = jnp.concatenate(ctx_parts, axis=1)
            out_ref[b, :, :] = jnp.dot(
                ctx, wo_ref[:, :], preferred_element_type=jnp.float32)

    return pl.pallas_call(
        body,
        out_shape=jax.ShapeDtypeStruct((B, S_LOC, D), jnp.float32),
        in_specs=[pl.BlockSpec(memory_space=pltpu.VMEM)] * 7,
        out_specs=pl.BlockSpec(memory_space=pltpu.VMEM),
        scratch_shapes=[
            pltpu.VMEM((B, S_LOC, HD), jnp.float32),
            pltpu.VMEM((B, S_GLOB, 2 * HD), jnp.float32),
            pltpu.VMEM((2, B, S_LOC, 2 * HD), jnp.float32),
            pltpu.SemaphoreType.DMA((2,)),
            pltpu.SemaphoreType.DMA((2,)),
        ],
        compiler_params=pltpu.CompilerParams(collective_id=0),
    )(x, Wq, Wk, Wv, Wo, cos_t, sin_t)


# device time: 20492 ns/iter; 1.7440x vs baseline; 1.7440x over previous
import jax
import jax.numpy as jnp
from jax import lax
from jax.experimental import pallas as pl
from jax.experimental.pallas import tpu as pltpu

N_DEV = 4
B = 2
S_LOC = 128
S_GLOB = N_DEV * S_LOC
D = 512
HQ = 4
DH = 64
HD = HQ * DH
SCALE = 0.125


def kernel(x, Wq, Wk, Wv, Wo):
    my = lax.axis_index("i")
    pos = (my * S_LOC + jnp.arange(S_LOC)).astype(jnp.float32)[:, None]
    inv = (1.0 / (10000.0 ** (jnp.arange(0, DH, 2).astype(jnp.float32) / DH)))
    ang = pos * inv[None, :]
    cos = jnp.repeat(jnp.cos(ang), 2, axis=-1)
    sin = jnp.repeat(jnp.sin(ang), 2, axis=-1)
    cos_t = jnp.tile(cos, (1, HQ))
    sin_t = jnp.tile(sin, (1, HQ))

    def body(x_ref, wq_ref, wk_ref, wv_ref, wo_ref, cos_ref, sin_ref,
             out_ref, kv_slots, send_sems, recv_sems):
        my_pos = lax.axis_index("i")

        barrier_sem = pltpu.get_barrier_semaphore()
        for d in (1, 2, 3):
            pl.semaphore_signal(barrier_sem, inc=1,
                                device_id=((my_pos + d) % N_DEV,),
                                device_id_type=pl.DeviceIdType.MESH)
        pl.semaphore_wait(barrier_sem, N_DEV - 1)

        cos_v = cos_ref[:, :]
        sin_v = sin_ref[:, :]
        lane = lax.broadcasted_iota(jnp.int32, (S_LOC, HD), 1)
        even = (lane % 2) == 0

        def rope(t):
            t_r = jnp.where(even, -jnp.roll(t, -1, axis=1),
                            jnp.roll(t, 1, axis=1))
            return t * cos_v + t_r * sin_v

        qs = []
        for b in range(B):
            xb = x_ref[b, :, :]
            q = jnp.dot(xb, wq_ref[:, :], preferred_element_type=jnp.float32)
            k = jnp.dot(xb, wk_ref[:, :], preferred_element_type=jnp.float32)
            v = jnp.dot(xb, wv_ref[:, :], preferred_element_type=jnp.float32)
            qs.append(rope(q).astype(jnp.bfloat16))
            kv_slots[my_pos, b, :, 0:HD] = rope(k).astype(jnp.bfloat16)
            kv_slots[my_pos, b, :, HD:2 * HD] = v.astype(jnp.bfloat16)

        sends = []
        for d in (1, 2, 3):
            rdma = pltpu.make_async_remote_copy(
                src_ref=kv_slots.at[my_pos],
                dst_ref=kv_slots.at[my_pos],
                send_sem=send_sems.at[d],
                recv_sem=recv_sems.at[my_pos],
                device_id=((my_pos + d) % N_DEV,),
                device_id_type=pl.DeviceIdType.MESH,
            )
            rdma.start()
            sends.append(rdma)

        s_blocks = [[None] * N_DEV for _ in range(B * HQ)]
        for o in range(N_DEV):
            @pl.when(o != my_pos)
            def _():
                recv = pltpu.make_async_remote_copy(
                    src_ref=kv_slots.at[o],
                    dst_ref=kv_slots.at[o],
                    send_sem=send_sems.at[0],
                    recv_sem=recv_sems.at[o],
                    device_id=(my_pos,),
                    device_id_type=pl.DeviceIdType.MESH,
                )
                recv.wait_recv()
            for b in range(B):
                k_o = kv_slots[o, b, :, 0:HD]
                for hh in range(HQ):
                    qh = qs[b][:, hh * DH:(hh + 1) * DH]
                    kh = k_o[:, hh * DH:(hh + 1) * DH]
                    s_blocks[b * HQ + hh][o] = lax.dot_general(
                        qh, kh, (((1,), (1,)), ((), ())),
                        preferred_element_type=jnp.float32)

        for b in range(B):
            ctx_parts = []
            for hh in range(HQ):
                s = jnp.concatenate(s_blocks[b * HQ + hh], axis=1) * SCALE
                m = jnp.max(s, axis=1, keepdims=True)
                w = jnp.exp(s - m)
                w = (w / jnp.sum(w, axis=1, keepdims=True)).astype(jnp.bfloat16)
                ctx = jnp.zeros((S_LOC, DH), jnp.float32)
                for o in range(N_DEV):
                    v_o = kv_slots[o, b, :, HD + hh * DH:HD + (hh + 1) * DH]
                    ctx += jnp.dot(w[:, o * S_LOC:(o + 1) * S_LOC], v_o,
                                   preferred_element_type=jnp.float32)
                ctx_parts.append(ctx)
            ctx_full = jnp.concatenate(ctx_parts, axis=1)
            out_ref[b, :, :] = jnp.dot(
                ctx_full, wo_ref[:, :], preferred_element_type=jnp.float32)

        for rdma in sends:
            rdma.wait_send()

    return pl.pallas_call(
        body,
        out_shape=jax.ShapeDtypeStruct((B, S_LOC, D), jnp.float32),
        in_specs=[pl.BlockSpec(memory_space=pltpu.VMEM)] * 7,
        out_specs=pl.BlockSpec(memory_space=pltpu.VMEM),
        scratch_shapes=[
            pltpu.VMEM((N_DEV, B, S_LOC, 2 * HD), jnp.bfloat16),
            pltpu.SemaphoreType.DMA((N_DEV,)),
            pltpu.SemaphoreType.DMA((N_DEV,)),
        ],
        compiler_params=pltpu.CompilerParams(collective_id=0),
    )(x, Wq, Wk, Wv, Wo, cos_t, sin_t)


# device time: 18041 ns/iter; 1.9809x vs baseline; 1.1359x over previous
import jax
import jax.numpy as jnp
from jax import lax
from jax.experimental import pallas as pl
from jax.experimental.pallas import tpu as pltpu

N_DEV = 4
B = 2
S_LOC = 128
S_GLOB = N_DEV * S_LOC
D = 512
HQ = 4
DH = 64
HD = HQ * DH
SCALE = 0.125


def kernel(x, Wq, Wk, Wv, Wo):
    my = lax.axis_index("i")
    pos = (my * S_LOC + jnp.arange(S_LOC)).astype(jnp.float32)[:, None]
    inv = (1.0 / (10000.0 ** (jnp.arange(0, DH, 2).astype(jnp.float32) / DH)))
    ang = pos * inv[None, :]
    cos = jnp.repeat(jnp.cos(ang), 2, axis=-1)
    sin = jnp.repeat(jnp.sin(ang), 2, axis=-1)
    cos_t = jnp.tile(cos, (1, HQ))
    sin_t = jnp.tile(sin, (1, HQ))

    def body(x_ref, wq_ref, wk_ref, wv_ref, wo_ref, cos_ref, sin_ref,
             out_ref, kv_slots, send_sems, recv_sems):
        my_pos = lax.axis_index("i")

        barrier_sem = pltpu.get_barrier_semaphore()
        for d in (1, 2, 3):
            pl.semaphore_signal(barrier_sem, inc=1,
                                device_id=((my_pos + d) % N_DEV,),
                                device_id_type=pl.DeviceIdType.MESH)
        pl.semaphore_wait(barrier_sem, N_DEV - 1)

        cos_v = cos_ref[:, :]
        sin_v = sin_ref[:, :]
        lane = lax.broadcasted_iota(jnp.int32, (S_LOC, HD), 1)
        even = (lane % 2) == 0

        def rope(t):
            t_r = jnp.where(even, -jnp.roll(t, -1, axis=1),
                            jnp.roll(t, 1, axis=1))
            return t * cos_v + t_r * sin_v

        xbs = [x_ref[b, :, :].astype(jnp.bfloat16) for b in range(B)]
        wk16 = wk_ref[:, :].astype(jnp.bfloat16)
        wv16 = wv_ref[:, :].astype(jnp.bfloat16)

        for b in range(B):
            k = jnp.dot(xbs[b], wk16, preferred_element_type=jnp.float32)
            v = jnp.dot(xbs[b], wv16, preferred_element_type=jnp.float32)
            kv_slots[my_pos, b, :, 0:HD] = rope(k).astype(jnp.bfloat16)
            kv_slots[my_pos, b, :, HD:2 * HD] = v.astype(jnp.bfloat16)

        sends = []
        for d in (1, 2, 3):
            rdma = pltpu.make_async_remote_copy(
                src_ref=kv_slots.at[my_pos],
                dst_ref=kv_slots.at[my_pos],
                send_sem=send_sems.at[d],
                recv_sem=recv_sems.at[my_pos],
                device_id=((my_pos + d) % N_DEV,),
                device_id_type=pl.DeviceIdType.MESH,
            )
            rdma.start()
            sends.append(rdma)

        wq16 = wq_ref[:, :].astype(jnp.bfloat16)
        qs = []
        for b in range(B):
            q = jnp.dot(xbs[b], wq16, preferred_element_type=jnp.float32)
            qs.append((rope(q) * SCALE).astype(jnp.bfloat16))

        s_blocks = [[None] * N_DEV for _ in range(B * HQ)]
        for o in range(N_DEV):
            @pl.when(o != my_pos)
            def _():
                recv = pltpu.make_async_remote_copy(
                    src_ref=kv_slots.at[o],
                    dst_ref=kv_slots.at[o],
                    send_sem=send_sems.at[0],
                    recv_sem=recv_sems.at[o],
                    device_id=(my_pos,),
                    device_id_type=pl.DeviceIdType.MESH,
                )
                recv.wait_recv()
            for b in range(B):
                k_o = kv_slots[o, b, :, 0:HD]
                for hh in range(HQ):
                    qh = qs[b][:, hh * DH:(hh + 1) * DH]
                    kh = k_o[:, hh * DH:(hh + 1) * DH]
                    s_blocks[b * HQ + hh][o] = lax.dot_general(
                        qh, kh, (((1,), (1,)), ((), ())),
                        preferred_element_type=jnp.float32)

        wo16 = wo_ref[:, :].astype(jnp.bfloat16)
        for b in range(B):
            ctx_parts = []
            for hh in range(HQ):
                s = jnp.concatenate(s_blocks[b * HQ + hh], axis=1)
                w = jnp.exp(s)
                r = 1.0 / jnp.sum(w, axis=1, keepdims=True)
                w16 = w.astype(jnp.bfloat16)
                ctx = jnp.zeros((S_LOC, DH), jnp.float32)
                for o in range(N_DEV):
                    v_o = kv_slots[o, b, :, HD + hh * DH:HD + (hh + 1) * DH]
                    ctx += jnp.dot(w16[:, o * S_LOC:(o + 1) * S_LOC], v_o,
                                   preferred_element_type=jnp.float32)
                ctx_parts.append(ctx * r)
            ctx_full = jnp.concatenate(ctx_parts, axis=1).astype(jnp.bfloat16)
            out_ref[b, :, :] = jnp.dot(
                ctx_full, wo16, preferred_element_type=jnp.float32)

        for rdma in sends:
            rdma.wait_send()

    return pl.pallas_call(
        body,
        out_shape=jax.ShapeDtypeStruct((B, S_LOC, D), jnp.float32),
        in_specs=[pl.BlockSpec(memory_space=pltpu.VMEM)] * 7,
        out_specs=pl.BlockSpec(memory_space=pltpu.VMEM),
        scratch_shapes=[
            pltpu.VMEM((N_DEV, B, S_LOC, 2 * HD), jnp.bfloat16),
            pltpu.SemaphoreType.DMA((N_DEV,)),
            pltpu.SemaphoreType.DMA((N_DEV,)),
        ],
        compiler_params=pltpu.CompilerParams(collective_id=0),
    )(x, Wq, Wk, Wv, Wo, cos_t, sin_t)
